# baseline (speedup 1.0000x reference)
import numpy as np

import jax
import jax.numpy as jnp
from jax.experimental import pallas as pl

B = 64
C = 768
H = 14
W = 14
HW = H * W
FREQ = 256
E = 64
K = 8
BB = 8
G = 24
GW = 6272
GC = 32

_S_NP = (np.arange(GW)[:, None] // HW == np.arange(GC)[None, :]).astype(
    np.float32) * np.float32(1.0 / HW)


def _routing_body(x_ref, freq_ref, seg_ref, wg_ref, wf_ref,
                  gates_ref, idx_ref, val_ref):
    pooled_g = jax.lax.dot_general(
        x_ref[...], seg_ref[...],
        dimension_numbers=(((1,), (0,)), ((), ())),
        preferred_element_type=jnp.float32,
    )

    p3 = pooled_g.reshape(BB, G, GC)
    logits = jax.lax.dot_general(
        freq_ref[...], wf_ref[...],
        dimension_numbers=(((1,), (1,)), ((), ())),
        preferred_element_type=jnp.float32,
    )
    for g in range(G):
        logits += jax.lax.dot_general(
            p3[:, g, :], wg_ref[g],
            dimension_numbers=(((1,), (0,)), ((), ())),
            preferred_element_type=jnp.float32,
        )

    m = jnp.max(logits, axis=-1, keepdims=True)
    ex = jnp.exp(logits - m)
    scores = ex / jnp.sum(ex, axis=-1, keepdims=True)

    iota = jax.lax.broadcasted_iota(jnp.int32, (BB, E), 1)
    active = jnp.ones((BB, E), dtype=jnp.bool_)
    gates = jnp.zeros((BB, E), dtype=jnp.float32)
    idxs = []
    vals = []
    for _ in range(K):
        masked = jnp.where(active, scores, -jnp.inf)
        v = jnp.max(masked, axis=-1, keepdims=True)
        cand = jnp.where(masked == v, iota, E)
        i = jnp.min(cand, axis=-1, keepdims=True)
        gates = jnp.where(iota == i, v, gates)
        active = active & (iota != i)
        idxs.append(i)
        vals.append(v)

    gates_ref[...] = gates
    idx_ref[...] = jnp.concatenate(idxs, axis=-1)
    val_ref[...] = jnp.concatenate(vals, axis=-1)


@jax.jit
def kernel(x, freq_emb, W_gate, W_freq):
    seg = jnp.asarray(_S_NP)
    wg3 = W_gate.T.reshape(G, GC, E)
    gates, idx, val = pl.pallas_call(
        _routing_body,
        grid=(B // BB,),
        in_specs=[
            pl.BlockSpec((BB * G, GW), lambda b: (b, 0)),
            pl.BlockSpec((BB, FREQ), lambda b: (b, 0)),
            pl.BlockSpec((GW, GC), lambda b: (0, 0)),
            pl.BlockSpec((G, GC, E), lambda b: (0, 0, 0)),
            pl.BlockSpec((E, FREQ), lambda b: (0, 0)),
        ],
        out_specs=[
            pl.BlockSpec((BB, E), lambda b: (b, 0)),
            pl.BlockSpec((BB, K), lambda b: (b, 0)),
            pl.BlockSpec((BB, K), lambda b: (b, 0)),
        ],
        out_shape=[
            jax.ShapeDtypeStruct((B, E), jnp.float32),
            jax.ShapeDtypeStruct((B, K), jnp.int32),
            jax.ShapeDtypeStruct((B, K), jnp.float32),
        ],
    )(x.reshape(B * G, GW), freq_emb, seg, wg3, W_freq)
    return gates, idx, val


# speedup vs baseline: 16.0082x; 16.0082x over previous
import jax
import jax.numpy as jnp
from jax.experimental import pallas as pl
from jax.experimental.pallas import tpu as pltpu

B = 64
C = 768
H = 14
W = 14
HW = H * W
FREQ = 256
E = 64
K = 8
S = 14
NSTEPS = HW // S


def _routing_body(x_ref, freq_ref, wg_ref, wf_ref,
                  gates_ref, idx_ref, val_ref, acc_ref):
    g = pl.program_id(0)

    @pl.when(g == 0)
    def _init():
        acc_ref[...] = jnp.zeros_like(acc_ref)

    acc_ref[...] += jnp.sum(x_ref[...], axis=0)

    @pl.when(g == NSTEPS - 1)
    def _epilogue():
        pooled = acc_ref[...] * (1.0 / HW)
        logits = jax.lax.dot_general(
            pooled, wg_ref[...],
            dimension_numbers=(((1,), (1,)), ((), ())),
            preferred_element_type=jnp.float32,
        )
        logits += jax.lax.dot_general(
            freq_ref[...], wf_ref[...],
            dimension_numbers=(((1,), (1,)), ((), ())),
            preferred_element_type=jnp.float32,
        )

        m = jnp.max(logits, axis=-1, keepdims=True)
        ex = jnp.exp(logits - m)
        scores = ex / jnp.sum(ex, axis=-1, keepdims=True)

        iota = jax.lax.broadcasted_iota(jnp.int32, (B, E), 1)
        active = jnp.ones((B, E), dtype=jnp.bool_)
        gates = jnp.zeros((B, E), dtype=jnp.float32)
        idxs = []
        vals = []
        for _ in range(K):
            masked = jnp.where(active, scores, -jnp.inf)
            v = jnp.max(masked, axis=-1, keepdims=True)
            cand = jnp.where(masked == v, iota, E)
            i = jnp.min(cand, axis=-1, keepdims=True)
            gates = jnp.where(iota == i, v, gates)
            active = active & (iota != i)
            idxs.append(i)
            vals.append(v)

        gates_ref[...] = gates
        idx_ref[...] = jnp.concatenate(idxs, axis=-1)
        val_ref[...] = jnp.concatenate(vals, axis=-1)


@jax.jit
def kernel(x, freq_emb, W_gate, W_freq):
    xt = jnp.transpose(x, (2, 3, 0, 1)).reshape(HW, B, C)
    gates, idx, val = pl.pallas_call(
        _routing_body,
        grid=(NSTEPS,),
        in_specs=[
            pl.BlockSpec((S, B, C), lambda g: (g, 0, 0)),
            pl.BlockSpec((B, FREQ), lambda g: (0, 0)),
            pl.BlockSpec((E, C), lambda g: (0, 0)),
            pl.BlockSpec((E, FREQ), lambda g: (0, 0)),
        ],
        out_specs=[
            pl.BlockSpec((B, E), lambda g: (0, 0)),
            pl.BlockSpec((B, K), lambda g: (0, 0)),
            pl.BlockSpec((B, K), lambda g: (0, 0)),
        ],
        out_shape=[
            jax.ShapeDtypeStruct((B, E), jnp.float32),
            jax.ShapeDtypeStruct((B, K), jnp.int32),
            jax.ShapeDtypeStruct((B, K), jnp.float32),
        ],
        scratch_shapes=[pltpu.VMEM((B, C), jnp.float32)],
    )(xt, freq_emb, W_gate, W_freq)
    return gates, idx, val


# speedup vs baseline: 18.1930x; 1.1365x over previous
import jax
import jax.numpy as jnp
from jax.experimental import pallas as pl
from jax.experimental.pallas import tpu as pltpu

B = 64
C = 768
H = 14
W = 14
HW = H * W
FREQ = 256
E = 64
K = 8
S = 28
NSTEPS = HW // S


def _routing_body(x_ref, freq_ref, wg_ref, wf_ref,
                  gates_ref, idx_ref, val_ref, acc_ref):
    g = pl.program_id(0)

    @pl.when(g == 0)
    def _init():
        acc_ref[...] = jnp.zeros_like(acc_ref)

    acc_ref[...] += jnp.sum(x_ref[...], axis=0)

    @pl.when(g == NSTEPS - 1)
    def _epilogue():
        pooled = acc_ref[...] * (1.0 / HW)
        logits = jax.lax.dot_general(
            pooled, wg_ref[...],
            dimension_numbers=(((1,), (1,)), ((), ())),
            preferred_element_type=jnp.float32,
        )
        logits += jax.lax.dot_general(
            freq_ref[...], wf_ref[...],
            dimension_numbers=(((1,), (1,)), ((), ())),
            preferred_element_type=jnp.float32,
        )

        m = jnp.max(logits, axis=-1, keepdims=True)
        ex = jnp.exp(logits - m)
        scores = ex / jnp.sum(ex, axis=-1, keepdims=True)

        iota = jax.lax.broadcasted_iota(jnp.int32, (B, E), 1)
        active = jnp.ones((B, E), dtype=jnp.bool_)
        gates = jnp.zeros((B, E), dtype=jnp.float32)
        idxs = []
        vals = []
        for _ in range(K):
            masked = jnp.where(active, scores, -jnp.inf)
            v = jnp.max(masked, axis=-1, keepdims=True)
            cand = jnp.where(masked == v, iota, E)
            i = jnp.min(cand, axis=-1, keepdims=True)
            gates = jnp.where(iota == i, v, gates)
            active = active & (iota != i)
            idxs.append(i)
            vals.append(v)

        gates_ref[...] = gates
        idx_ref[...] = jnp.concatenate(idxs, axis=-1)
        val_ref[...] = jnp.concatenate(vals, axis=-1)


@jax.jit
def kernel(x, freq_emb, W_gate, W_freq):
    xt = jnp.transpose(x, (2, 3, 0, 1)).reshape(HW, B, C)
    gates, idx, val = pl.pallas_call(
        _routing_body,
        grid=(NSTEPS,),
        in_specs=[
            pl.BlockSpec((S, B, C), lambda g: (g, 0, 0)),
            pl.BlockSpec((B, FREQ), lambda g: (0, 0)),
            pl.BlockSpec((E, C), lambda g: (0, 0)),
            pl.BlockSpec((E, FREQ), lambda g: (0, 0)),
        ],
        out_specs=[
            pl.BlockSpec((B, E), lambda g: (0, 0)),
            pl.BlockSpec((B, K), lambda g: (0, 0)),
            pl.BlockSpec((B, K), lambda g: (0, 0)),
        ],
        out_shape=[
            jax.ShapeDtypeStruct((B, E), jnp.float32),
            jax.ShapeDtypeStruct((B, K), jnp.int32),
            jax.ShapeDtypeStruct((B, K), jnp.float32),
        ],
        scratch_shapes=[pltpu.VMEM((B, C), jnp.float32)],
    )(xt, freq_emb, W_gate, W_freq)
    return gates, idx, val
